# baseline (speedup 1.0000x reference)
import functools

import jax
import jax.numpy as jnp
from jax import lax
from jax.experimental import pallas as pl
from jax.experimental.pallas import tpu as pltpu
from jax.experimental.pallas import tpu_sc as plsc

_NC, _NS = 2, 16
_NW = _NC * _NS


def _make_sc_gather(R, D, chunk):
    r_per_w = R // _NW
    n_chunks = r_per_w // chunk
    mesh = plsc.VectorSubcoreMesh(
        core_axis_name="c", subcore_axis_name="s",
        num_cores=_NC, num_subcores=_NS,
    )

    @functools.partial(
        pl.kernel,
        mesh=mesh,
        out_type=jax.ShapeDtypeStruct((R, D), jnp.float32),
        scratch_types=[
            pltpu.VMEM((chunk,), jnp.int32),
            pltpu.VMEM((chunk, D), jnp.float32),
            pltpu.SemaphoreType.DMA,
        ],
    )
    def gather(table_hbm, idx_hbm, out_hbm, idx_v, rows_v, sem):
        wid = lax.axis_index("s") * _NC + lax.axis_index("c")
        base = wid * r_per_w

        def body(c, carry):
            off = base + c * chunk
            pltpu.sync_copy(idx_hbm.at[pl.ds(off, chunk)], idx_v)
            pltpu.async_copy(table_hbm.at[idx_v], rows_v, sem).wait()
            pltpu.sync_copy(rows_v, out_hbm.at[pl.ds(off, chunk)])
            return carry

        jax.lax.fori_loop(0, n_chunks, body, 0)

    return gather


def _mm_kernel(xg_ref, w_ref, out_ref):
    out_ref[0, 0] = jnp.dot(
        xg_ref[0, 0], w_ref[0], preferred_element_type=jnp.float32
    )


@jax.jit
def _run(x, indices, W):
    B, T, I = x.shape
    _, E, K = indices.shape
    J = W.shape[2]

    x2 = x.reshape(B * T, I)
    idx2 = indices + (jnp.arange(B, dtype=jnp.int32) * T)[:, None, None]
    idx_all = idx2.transpose(1, 0, 2).reshape(E * B * K)

    xg = _make_sc_gather(E * B * K, I, 32)(x2, idx_all)
    xg = xg.reshape(E, B, K, I)

    mm = pl.pallas_call(
        _mm_kernel,
        grid=(E, B),
        in_specs=[
            pl.BlockSpec((1, 1, K, I), lambda e, b: (e, b, 0, 0)),
            pl.BlockSpec((1, I, J), lambda e, b: (e, 0, 0)),
        ],
        out_specs=pl.BlockSpec((1, 1, K, J), lambda e, b: (b, e, 0, 0)),
        out_shape=jax.ShapeDtypeStruct((B, E, K, J), jnp.float32),
        compiler_params=pltpu.CompilerParams(
            dimension_semantics=("arbitrary", "arbitrary"),
        ),
    )
    return mm(xg, W)


def kernel(x, indices, W):
    return _run(x, indices, W)


# speedup vs baseline: 1.1021x; 1.1021x over previous
import functools

import jax
import jax.numpy as jnp
from jax import lax
from jax.experimental import pallas as pl
from jax.experimental.pallas import tpu as pltpu
from jax.experimental.pallas import tpu_sc as plsc

_NC, _NS = 2, 16
_NW = _NC * _NS
_EC = 2


def _make_sc_gather(R, D, chunk):
    r_per_w = R // _NW
    n_chunks = r_per_w // chunk
    mesh = plsc.VectorSubcoreMesh(
        core_axis_name="c", subcore_axis_name="s",
        num_cores=_NC, num_subcores=_NS,
    )

    @functools.partial(
        pl.kernel,
        mesh=mesh,
        out_type=jax.ShapeDtypeStruct((R, D), jnp.float32),
        scratch_types=[
            pltpu.VMEM((chunk,), jnp.int32),
            pltpu.VMEM((chunk, D), jnp.float32),
            pltpu.SemaphoreType.DMA,
        ],
    )
    def gather(table_hbm, idx_hbm, out_hbm, idx_v, rows_v, sem):
        wid = lax.axis_index("s") * _NC + lax.axis_index("c")
        base = wid * r_per_w

        def body(c, carry):
            off = base + c * chunk
            pltpu.sync_copy(idx_hbm.at[pl.ds(off, chunk)], idx_v)
            pltpu.async_copy(table_hbm.at[idx_v], rows_v, sem).wait()
            pltpu.sync_copy(rows_v, out_hbm.at[pl.ds(off, chunk)])
            return carry

        jax.lax.fori_loop(0, n_chunks, body, 0)

    return gather


def _mm_kernel(xg_ref, w_ref, ydon_ref, out_ref):
    del ydon_ref
    out_ref[0, 0] = jnp.dot(
        xg_ref[0, 0], w_ref[0], preferred_element_type=jnp.float32
    )


@jax.jit
def _run(x, indices, W):
    B, T, I = x.shape
    _, E, K = indices.shape
    J = W.shape[2]

    x2 = x.reshape(B * T, I)
    idx2 = indices + (jnp.arange(B, dtype=jnp.int32) * T)[:, None, None]
    idx_all = idx2.transpose(1, 0, 2).reshape(E * B * K)

    sc_gather = _make_sc_gather(_EC * B * K, I, 32)
    n_chunks = E // _EC

    def make_mm(c, aliased):
        return pl.pallas_call(
            _mm_kernel,
            grid=(_EC, B),
            in_specs=[
                pl.BlockSpec((1, 1, K, I), lambda ec, b: (ec, b, 0, 0)),
                pl.BlockSpec((1, I, J), lambda ec, b: (c * _EC + ec, 0, 0)),
                pl.BlockSpec(memory_space=pl.ANY),
            ],
            out_specs=pl.BlockSpec(
                (1, 1, K, J), lambda ec, b: (b, c * _EC + ec, 0, 0)
            ),
            out_shape=jax.ShapeDtypeStruct((B, E, K, J), jnp.float32),
            compiler_params=pltpu.CompilerParams(
                dimension_semantics=("arbitrary", "arbitrary"),
            ),
            input_output_aliases={2: 0} if aliased else {},
        )

    xgs = [
        sc_gather(x2, lax.slice(idx_all, (c * _EC * B * K,),
                                ((c + 1) * _EC * B * K,)))
        .reshape(_EC, B, K, I)
        for c in range(n_chunks)
    ]

    y = jnp.zeros((8,), jnp.float32)
    for c in range(n_chunks):
        y = make_mm(c, aliased=(c > 0))(xgs[c], W, y)
    return y


def kernel(x, indices, W):
    return _run(x, indices, W)


# speedup vs baseline: 1.1038x; 1.0016x over previous
import functools

import jax
import jax.numpy as jnp
from jax import lax
from jax.experimental import pallas as pl
from jax.experimental.pallas import tpu as pltpu
from jax.experimental.pallas import tpu_sc as plsc

_NC, _NS = 2, 16
_NW = _NC * _NS
_EC = 1


def _make_sc_gather(R, D, chunk):
    r_per_w = R // _NW
    n_chunks = r_per_w // chunk
    mesh = plsc.VectorSubcoreMesh(
        core_axis_name="c", subcore_axis_name="s",
        num_cores=_NC, num_subcores=_NS,
    )

    @functools.partial(
        pl.kernel,
        mesh=mesh,
        out_type=jax.ShapeDtypeStruct((R, D), jnp.float32),
        scratch_types=[
            pltpu.VMEM((chunk,), jnp.int32),
            pltpu.VMEM((chunk, D), jnp.float32),
            pltpu.SemaphoreType.DMA,
        ],
    )
    def gather(table_hbm, idx_hbm, out_hbm, idx_v, rows_v, sem):
        wid = lax.axis_index("s") * _NC + lax.axis_index("c")
        base = wid * r_per_w

        def body(c, carry):
            off = base + c * chunk
            pltpu.sync_copy(idx_hbm.at[pl.ds(off, chunk)], idx_v)
            pltpu.async_copy(table_hbm.at[idx_v], rows_v, sem).wait()
            pltpu.sync_copy(rows_v, out_hbm.at[pl.ds(off, chunk)])
            return carry

        jax.lax.fori_loop(0, n_chunks, body, 0)

    return gather


def _mm_kernel(xg_ref, w_ref, ydon_ref, out_ref):
    del ydon_ref
    out_ref[0, 0] = jnp.dot(
        xg_ref[0, 0], w_ref[0], preferred_element_type=jnp.float32
    )


@jax.jit
def _run(x, indices, W):
    B, T, I = x.shape
    _, E, K = indices.shape
    J = W.shape[2]

    x2 = x.reshape(B * T, I)
    idx2 = indices + (jnp.arange(B, dtype=jnp.int32) * T)[:, None, None]
    idx_all = idx2.transpose(1, 0, 2).reshape(E * B * K)

    sc_gather = _make_sc_gather(_EC * B * K, I, 32)
    n_chunks = E // _EC

    def make_mm(c, aliased):
        return pl.pallas_call(
            _mm_kernel,
            grid=(_EC, B),
            in_specs=[
                pl.BlockSpec((1, 1, K, I), lambda ec, b: (ec, b, 0, 0)),
                pl.BlockSpec((1, I, J), lambda ec, b: (c * _EC + ec, 0, 0)),
                pl.BlockSpec(memory_space=pl.ANY),
            ],
            out_specs=pl.BlockSpec(
                (1, 1, K, J), lambda ec, b: (b, c * _EC + ec, 0, 0)
            ),
            out_shape=jax.ShapeDtypeStruct((B, E, K, J), jnp.float32),
            compiler_params=pltpu.CompilerParams(
                dimension_semantics=("arbitrary", "arbitrary"),
            ),
            input_output_aliases={2: 0} if aliased else {},
        )

    xgs = [
        sc_gather(x2, lax.slice(idx_all, (c * _EC * B * K,),
                                ((c + 1) * _EC * B * K,)))
        .reshape(_EC, B, K, I)
        for c in range(n_chunks)
    ]

    y = jnp.zeros((8,), jnp.float32)
    for c in range(n_chunks):
        y = make_mm(c, aliased=(c > 0))(xgs[c], W, y)
    return y


def kernel(x, indices, W):
    return _run(x, indices, W)
